# baseline (speedup 1.0000x reference)
import jax
import jax.numpy as jnp
import numpy as np
from jax.experimental import pallas as pl

B = 16384
S = 16
P = 16
EPS = 1e-8
ACC_STATE = 15
BLOCK_B = 16384
NSLAB = 11


def _automaton_constants():
    rng = np.random.RandomState(0)
    exists = (rng.rand(S, S) < 0.35).astype(np.float32)
    exists[np.arange(S), (np.arange(S) + 1) % S] = 1.0
    mask = (rng.rand(S, S, P) < 0.2).astype(np.float32) * exists[:, :, None]
    m2t = np.zeros((NSLAB * S, P), dtype=np.float32)
    selt = np.zeros((NSLAB * S, S), dtype=np.float32)
    for s1 in range(S):
        srcs = np.nonzero(exists[:, s1])[0]
        assert len(srcs) <= NSLAB
        for k, s0 in enumerate(srcs):
            m2t[k * S + s1] = mask[s0, s1]
            selt[k * S + s1, s0] = 1.0
    return jnp.asarray(m2t), jnp.asarray(selt)


def _fused_kernel(c_ref, s0_ref, m2t_ref, selt_ref, ln_ref, ns_ref, acc_ref):
    f32 = jnp.float32
    dn = (((1,), (1,)), ((), ()))
    lc = jnp.log(c_ref[:] + 1e-12)
    tl = jax.lax.dot_general(m2t_ref[:], lc, dn,
                             preferred_element_type=f32)
    gate = jax.lax.dot_general(selt_ref[:], s0_ref[:], dn,
                               preferred_element_type=f32)
    proofs = jnp.exp(tl) * gate

    neg = jnp.full((S, proofs.shape[1]), -1.0, dtype=f32)
    m1, m2, m3 = neg, neg, neg
    for k in range(NSLAB):
        v = proofs[k * S:(k + 1) * S, :]
        nm1 = jnp.maximum(m1, v)
        t = jnp.minimum(m1, v)
        nm2 = jnp.maximum(m2, t)
        t2 = jnp.minimum(m2, t)
        m3 = jnp.maximum(m3, t2)
        m1, m2 = nm1, nm2
    total = m1 + m2 + m3

    denom = jnp.sum(total, axis=0, keepdims=True) + EPS
    nxt = total / denom
    eye = jnp.eye(S, dtype=f32)
    dt = (((0,), (0,)), ((), ()))
    ln_ref[:] = jax.lax.dot_general(jnp.log(nxt + EPS), eye, dt,
                                    preferred_element_type=f32)
    ns_ref[:] = jax.lax.dot_general(nxt, eye, dt,
                                    preferred_element_type=f32)
    acc_ref[0, :] = total[ACC_STATE]


def kernel(log_s0, s0, constraints):
    del log_s0
    m2t, selt = _automaton_constants()
    grid = (B // BLOCK_B,)
    ln, ns, acc = pl.pallas_call(
        _fused_kernel,
        grid=grid,
        in_specs=[
            pl.BlockSpec((BLOCK_B, P), lambda i: (i, 0)),
            pl.BlockSpec((BLOCK_B, S), lambda i: (i, 0)),
            pl.BlockSpec((NSLAB * S, P), lambda i: (0, 0)),
            pl.BlockSpec((NSLAB * S, S), lambda i: (0, 0)),
        ],
        out_specs=[
            pl.BlockSpec((BLOCK_B, S), lambda i: (i, 0)),
            pl.BlockSpec((BLOCK_B, S), lambda i: (i, 0)),
            pl.BlockSpec((1, BLOCK_B), lambda i: (0, i)),
        ],
        out_shape=[
            jax.ShapeDtypeStruct((B, S), jnp.float32),
            jax.ShapeDtypeStruct((B, S), jnp.float32),
            jax.ShapeDtypeStruct((1, B), jnp.float32),
        ],
    )(constraints, s0, m2t, selt)
    return (ln, ns, acc.reshape(B))

def _probe_kernel(c_ref, s0_ref, ln_ref, ns_ref, acc_ref):
    ln_ref[:] = c_ref[:]
    ns_ref[:] = s0_ref[:]
    acc_ref[0, :] = c_ref[:, 0]


def _probe_call(log_s0, s0, constraints):
    grid = (B // BLOCK_B,)
    ln, ns, acc = pl.pallas_call(
        _probe_kernel,
        grid=grid,
        in_specs=[
            pl.BlockSpec((BLOCK_B, P), lambda i: (i, 0)),
            pl.BlockSpec((BLOCK_B, S), lambda i: (i, 0)),
        ],
        out_specs=[
            pl.BlockSpec((BLOCK_B, S), lambda i: (i, 0)),
            pl.BlockSpec((BLOCK_B, S), lambda i: (i, 0)),
            pl.BlockSpec((1, BLOCK_B), lambda i: (0, i)),
        ],
        out_shape=[
            jax.ShapeDtypeStruct((B, S), jnp.float32),
            jax.ShapeDtypeStruct((B, S), jnp.float32),
            jax.ShapeDtypeStruct((1, B), jnp.float32),
        ],
    )(constraints, s0)
    return (ln, ns, acc.reshape(B))

kernel = _probe_call


# speedup vs baseline: 1.0754x; 1.0754x over previous
import jax
import jax.numpy as jnp
import numpy as np
from jax.experimental import pallas as pl

B = 16384
S = 16
P = 16
EPS = 1e-8
ACC_STATE = 15
BLOCK_B = 4096
NSLAB = 11


def _automaton_constants():
    rng = np.random.RandomState(0)
    exists = (rng.rand(S, S) < 0.35).astype(np.float32)
    exists[np.arange(S), (np.arange(S) + 1) % S] = 1.0
    mask = (rng.rand(S, S, P) < 0.2).astype(np.float32) * exists[:, :, None]
    m2t = np.zeros((NSLAB * S, P), dtype=np.float32)
    selt = np.zeros((NSLAB * S, S), dtype=np.float32)
    for s1 in range(S):
        srcs = np.nonzero(exists[:, s1])[0]
        assert len(srcs) <= NSLAB
        for k, s0 in enumerate(srcs):
            m2t[k * S + s1] = mask[s0, s1]
            selt[k * S + s1, s0] = 1.0
    return jnp.asarray(m2t), jnp.asarray(selt)


def _fused_kernel(c_ref, s0_ref, m2t_ref, selt_ref, ln_ref, ns_ref, acc_ref):
    f32 = jnp.float32
    dn = (((1,), (1,)), ((), ()))
    lc = jnp.log(c_ref[:] + 1e-12)
    tl = jax.lax.dot_general(m2t_ref[:], lc, dn,
                             preferred_element_type=f32)
    gate = jax.lax.dot_general(selt_ref[:], s0_ref[:], dn,
                               preferred_element_type=f32)
    proofs = jnp.exp(tl) * gate

    v0 = proofs[0:S, :]
    v1 = proofs[S:2 * S, :]
    m1 = jnp.maximum(v0, v1)
    m2 = jnp.minimum(v0, v1)
    m3 = jnp.full((S, proofs.shape[1]), -1.0, dtype=f32)
    for k in range(2, NSLAB):
        v = proofs[k * S:(k + 1) * S, :]
        nm1 = jnp.maximum(m1, v)
        t = jnp.minimum(m1, v)
        nm2 = jnp.maximum(m2, t)
        t2 = jnp.minimum(m2, t)
        m3 = jnp.maximum(m3, t2)
        m1, m2 = nm1, nm2
    total = m1 + m2 + m3

    denom = jnp.sum(total, axis=0, keepdims=True) + EPS
    nxt = total / denom
    eye = jnp.eye(S, dtype=f32)
    dt = (((0,), (0,)), ((), ()))
    ln_ref[:] = jax.lax.dot_general(jnp.log(nxt + EPS), eye, dt,
                                    preferred_element_type=f32)
    ns_ref[:] = jax.lax.dot_general(nxt, eye, dt,
                                    preferred_element_type=f32)
    acc_ref[0, :] = total[ACC_STATE]


def kernel(log_s0, s0, constraints):
    del log_s0
    m2t, selt = _automaton_constants()
    grid = (B // BLOCK_B,)
    ln, ns, acc = pl.pallas_call(
        _fused_kernel,
        grid=grid,
        in_specs=[
            pl.BlockSpec((BLOCK_B, P), lambda i: (i, 0)),
            pl.BlockSpec((BLOCK_B, S), lambda i: (i, 0)),
            pl.BlockSpec((NSLAB * S, P), lambda i: (0, 0)),
            pl.BlockSpec((NSLAB * S, S), lambda i: (0, 0)),
        ],
        out_specs=[
            pl.BlockSpec((BLOCK_B, S), lambda i: (i, 0)),
            pl.BlockSpec((BLOCK_B, S), lambda i: (i, 0)),
            pl.BlockSpec((1, BLOCK_B), lambda i: (0, i)),
        ],
        out_shape=[
            jax.ShapeDtypeStruct((B, S), jnp.float32),
            jax.ShapeDtypeStruct((B, S), jnp.float32),
            jax.ShapeDtypeStruct((1, B), jnp.float32),
        ],
    )(constraints, s0, m2t, selt)
    return (ln, ns, acc.reshape(B))
